# baseline (speedup 1.0000x reference)
import functools

import jax
import jax.numpy as jnp
from jax import lax
from jax.experimental import pallas as pl
from jax.experimental.pallas import tpu as pltpu
from jax.experimental.pallas import tpu_sc as plsc

D = 512
L = 16
S = 50
SP = 56
NBUF = 2


def _sc_add_emb(x, idx_pad, table):
    B = x.shape[0]
    info = plsc.get_sparse_core_info()
    NC, NS = info.num_cores, info.num_subcores
    NW = NC * NS
    n_wb = B // NW
    mesh = plsc.VectorSubcoreMesh(core_axis_name="c", subcore_axis_name="s")

    @functools.partial(
        pl.kernel,
        mesh=mesh,
        out_type=jax.ShapeDtypeStruct((B, S, D), jnp.float32),
        scratch_types=[
            pltpu.VMEM((n_wb, SP), jnp.int32),
            pltpu.VMEM((NBUF, S, D), jnp.float32),
            pltpu.VMEM((NBUF, SP, D), jnp.float32),
            pltpu.SemaphoreType.DMA((NBUF,)),
            pltpu.SemaphoreType.DMA((NBUF,)),
            pltpu.SemaphoreType.DMA((NBUF,)),
        ],
    )
    def k(x_hbm, idx_hbm, tab_hbm, out_hbm,
          idx_all, x_v, rows_v, sem_x, sem_g, sem_o):
        wid = lax.axis_index("s") * NC + lax.axis_index("c")
        wb = wid * n_wb
        pltpu.sync_copy(idx_hbm.at[pl.ds(wb, n_wb)], idx_all)

        def in_copies(g, b):
            return (
                pltpu.make_async_copy(
                    x_hbm.at[wb + g], x_v.at[b], sem_x.at[b]),
                pltpu.make_async_copy(
                    tab_hbm.at[idx_all.at[g]], rows_v.at[b], sem_g.at[b]),
            )

        def out_copy(g, b):
            return pltpu.make_async_copy(
                x_v.at[b], out_hbm.at[wb + g], sem_o.at[b])

        def issue_in(g, b):
            for cp in in_copies(g, b):
                cp.start()

        issue_in(0, 0)

        def body(g, carry):
            b = lax.rem(g, NBUF)
            for cp in in_copies(g, b):
                cp.wait()

            def row(r, carry2):
                for j in range(D // L):
                    plsc.addupdate(
                        x_v.at[b, r, pl.ds(j * L, L)],
                        rows_v[b, r, pl.ds(j * L, L)],
                    )
                return carry2

            lax.fori_loop(0, S, row, 0)
            out_copy(g, b).start()

            g1 = g + 1
            b1 = lax.rem(g1, NBUF)

            @pl.when(jnp.logical_and(g >= 1, g1 < n_wb))
            def _():
                out_copy(g - 1, b1).wait()

            @pl.when(g1 < n_wb)
            def _():
                issue_in(g1, b1)

            return carry

        lax.fori_loop(0, n_wb, body, 0)

        for gd in range(n_wb - NBUF, n_wb):
            out_copy(gd, gd % NBUF).wait()

    return k(x, idx_pad, table)


def kernel(x, instruction_types, type_emb_weight):
    idx = instruction_types.astype(jnp.int32)
    idx_pad = jnp.pad(idx, ((0, 0), (0, SP - S)))
    return _sc_add_emb(x, idx_pad, type_emb_weight)
